# Initial kernel scaffold; baseline (speedup 1.0000x reference)
import jax
import jax.numpy as jnp
from jax.experimental import pallas as pl


def kernel(protein_embeddings, edge_index, W_in, b_in, W_l, W_r, att, bias_gat, W1, b1, W2, b2, W3, b3):
    raise NotImplementedError("write your pallas kernel here")


# speedup vs baseline: 3.5249x; 3.5249x over previous
import dataclasses
import functools

import jax
import jax.numpy as jnp
from jax import lax
from jax.experimental import pallas as pl
from jax.experimental.pallas import tpu as pltpu
from jax.experimental.pallas import tpu_sc as plsc

N = 1024
B = 16
IN = 128
C = 32
H = 4
E = 16384
BH = B * H
NEG_GAT = 0.2
NEG_PRE = 0.01

NW = 32
ECHUNK = 8192
EGROUP = 16

F32 = jnp.float32


_NBLK_A = 16


def _pergene_body(pe_ref, win_ref, bin_ref, x_ref):
    for j in range(_NBLK_A):
        a = pe_ref[j]
        w = win_ref[j]
        t = jnp.dot(a, w, preferred_element_type=F32) + bin_ref[j][None, :]
        x_ref[:, j, :] = jnp.maximum(t, NEG_PRE * t)


def _pergene(pe, w_in, b_in):
    grid = (N // _NBLK_A,)
    return pl.pallas_call(
        _pergene_body,
        grid=grid,
        in_specs=[
            pl.BlockSpec((_NBLK_A, B, IN), lambda i: (i, 0, 0)),
            pl.BlockSpec((_NBLK_A, IN, C), lambda i: (i, 0, 0)),
            pl.BlockSpec((_NBLK_A, C), lambda i: (i, 0)),
        ],
        out_specs=pl.BlockSpec((B, _NBLK_A, C), lambda i: (0, i, 0)),
        out_shape=jax.ShapeDtypeStruct((B, N, C), F32),
    )(pe, w_in, b_in)


def _proj_body(x_ref, wl_ref, wr_ref, xl_ref, xr_ref):
    xm = x_ref[0]
    rl = jnp.dot(xm, wl_ref[...], preferred_element_type=F32)
    rr = jnp.dot(xm, wr_ref[...], preferred_element_type=F32)
    for h in range(H):
        xl_ref[h] = rl[:, h * C:(h + 1) * C]
        xr_ref[h] = rr[:, h * C:(h + 1) * C]


def _project(x, w_l, w_r):
    grid = (B,)
    out_shape = [
        jax.ShapeDtypeStruct((BH, N, C), F32),
        jax.ShapeDtypeStruct((BH, N, C), F32),
    ]
    return pl.pallas_call(
        _proj_body,
        grid=grid,
        in_specs=[
            pl.BlockSpec((1, N, C), lambda b: (b, 0, 0)),
            pl.BlockSpec((C, H * C), lambda b: (0, 0)),
            pl.BlockSpec((C, H * C), lambda b: (0, 0)),
        ],
        out_specs=[
            pl.BlockSpec((H, N, C), lambda b: (b, 0, 0)),
            pl.BlockSpec((H, N, C), lambda b: (b, 0, 0)),
        ],
        out_shape=out_shape,
    )(x, w_l, w_r)


def _sc_edge_body(xlp, xrp, src, dst, attx, un_hbm, dn_hbm,
                  xl_v, xr_v, att_v, src_v, dst_v, un_v, dn_v):
    wid = lax.axis_index("s") * 2 + lax.axis_index("c")
    zero16 = jnp.zeros((EGROUP,), F32)

    for r in range(2):
        bh = wid * 2 + r
        h = bh % H

        pltpu.sync_copy(xlp.at[bh], xl_v)
        pltpu.sync_copy(xrp.at[bh], xr_v)
        pltpu.sync_copy(attx.at[h], att_v)

        @pl.loop(0, N * C // 16)
        def _zero_un(i):
            un_v[pl.ds(i * 16, 16)] = zero16

        @pl.loop(0, N // 16)
        def _zero_dn(i):
            dn_v[pl.ds(i * 16, 16)] = zero16

        for chunk in range(E // ECHUNK):
            pltpu.sync_copy(src.at[pl.ds(chunk * ECHUNK, ECHUNK)], src_v)
            pltpu.sync_copy(dst.at[pl.ds(chunk * ECHUNK, ECHUNK)], dst_v)

            @pl.loop(0, ECHUNK // EGROUP)
            def _edges(g):
                sv = src_v[pl.ds(g * EGROUP, EGROUP)]
                dv = dst_v[pl.ds(g * EGROUP, EGROUP)]
                sb = sv * C
                db = dv * C
                logit = zero16
                xs = []
                for c in range(C):
                    a = plsc.load_gather(xl_v, [sb + c])
                    bb = plsc.load_gather(xr_v, [db + c])
                    u = a + bb
                    lr = jnp.maximum(u, NEG_GAT * u)
                    logit = logit + att_v[pl.ds(c * 16, 16)] * lr
                    xs.append(a)
                ex = jnp.exp(logit)
                plsc.addupdate_scatter(dn_v, [dv], ex)
                for c in range(C):
                    plsc.addupdate_scatter(un_v, [db + c], ex * xs[c])

        pltpu.sync_copy(un_v, un_hbm.at[bh])
        pltpu.sync_copy(dn_v, dn_hbm.at[bh])


def _sc_edge(xlp, xrp, src, dst, attx):
    mesh = plsc.VectorSubcoreMesh(core_axis_name="c", subcore_axis_name="s",
                                  num_cores=2, num_subcores=16)
    cp = pltpu.CompilerParams()
    if "needs_layout_passes" in pltpu.CompilerParams.__dataclass_fields__:
        cp = dataclasses.replace(cp, needs_layout_passes=False)
    kern = pl.kernel(
        _sc_edge_body,
        out_type=[
            jax.ShapeDtypeStruct((BH, N * C), F32),
            jax.ShapeDtypeStruct((BH, N), F32),
        ],
        mesh=mesh,
        scratch_types=[
            pltpu.VMEM((N * C,), F32),
            pltpu.VMEM((N * C,), F32),
            pltpu.VMEM((C * 16,), F32),
            pltpu.VMEM((ECHUNK,), jnp.int32),
            pltpu.VMEM((ECHUNK,), jnp.int32),
            pltpu.VMEM((N * C,), F32),
            pltpu.VMEM((N,), F32),
        ],
        compiler_params=cp,
    )
    return kern(xlp, xrp, src, dst, attx)


_NBLK_C = 128


def _norm_body(un_ref, dn_ref, bias_ref, out_ref):
    a = un_ref[...].reshape(B, H, _NBLK_C, C)
    d = dn_ref[...].reshape(B, H, _NBLK_C)
    nrm = a / (d[..., None] + 1e-16)
    out_ref[...] = jnp.mean(nrm, axis=1) + bias_ref[0][None, None, :]


def _normalize(unnorm, denom, bias_gat):
    grid = (N // _NBLK_C,)
    return pl.pallas_call(
        _norm_body,
        grid=grid,
        in_specs=[
            pl.BlockSpec((BH, _NBLK_C, C), lambda i: (0, i, 0)),
            pl.BlockSpec((BH, _NBLK_C), lambda i: (0, i)),
            pl.BlockSpec((1, C), lambda i: (0, 0)),
        ],
        out_specs=pl.BlockSpec((B, _NBLK_C, C), lambda i: (0, i, 0)),
        out_shape=jax.ShapeDtypeStruct((B, N, C), F32),
    )(unnorm, denom, bias_gat)


_KBLK = 4096


def _mlp_body(flat_ref, w1_ref, b1_ref, w2_ref, b2_ref, w3_ref, b3_ref,
              out_ref, acc_ref):
    k = pl.program_id(0)

    @pl.when(k == 0)
    def _():
        acc_ref[...] = jnp.zeros_like(acc_ref)

    acc_ref[...] += jnp.dot(flat_ref[...], w1_ref[...],
                            preferred_element_type=F32)

    @pl.when(k == (N * C // _KBLK) - 1)
    def _():
        h1 = jnp.maximum(acc_ref[...] + b1_ref[...], 0.0)
        h2 = jnp.maximum(
            jnp.dot(h1, w2_ref[...], preferred_element_type=F32)
            + b2_ref[...], 0.0)
        out_ref[...] = (jnp.dot(h2, w3_ref[...], preferred_element_type=F32)
                        + b3_ref[...])


def _mlp(flat, w1, b1, w2, b2, w3, b3):
    grid = (N * C // _KBLK,)
    return pl.pallas_call(
        _mlp_body,
        grid=grid,
        in_specs=[
            pl.BlockSpec((B, _KBLK), lambda k: (0, k)),
            pl.BlockSpec((_KBLK, 256), lambda k: (k, 0)),
            pl.BlockSpec((1, 256), lambda k: (0, 0)),
            pl.BlockSpec((256, 64), lambda k: (0, 0)),
            pl.BlockSpec((1, 64), lambda k: (0, 0)),
            pl.BlockSpec((64, 1), lambda k: (0, 0)),
            pl.BlockSpec((1, 1), lambda k: (0, 0)),
        ],
        out_specs=pl.BlockSpec((B, 1), lambda k: (0, 0)),
        out_shape=jax.ShapeDtypeStruct((B, 1), F32),
        scratch_shapes=[pltpu.VMEM((B, 256), F32)],
    )(flat, w1, b1, w2, b2, w3, b3)


@jax.jit
def _run(pe, edge_index, w_in, b_in, w_l, w_r, att, bias_gat,
         w1, b1, w2, b2, w3, b3):
    src = edge_index[0]
    dst = edge_index[1]
    attx = jnp.broadcast_to(att[:, :, None], (H, C, 16)).astype(F32)
    attx = attx.reshape(H, C * 16)

    x = _pergene(pe, w_in, b_in)
    xlp, xrp = _project(x, w_l, w_r)
    unnorm, denom = _sc_edge(xlp.reshape(BH, N * C), xrp.reshape(BH, N * C),
                             src, dst, attx)
    agg = _normalize(unnorm.reshape(BH, N, C), denom, bias_gat.reshape(1, C))
    flat = agg.reshape(B, N * C)
    return _mlp(flat, w1, b1.reshape(1, 256), w2, b2.reshape(1, 64),
                w3, b3.reshape(1, 1))


def kernel(protein_embeddings, edge_index, W_in, b_in, W_l, W_r, att,
           bias_gat, W1, b1, W2, b2, W3, b3):
    return _run(protein_embeddings, edge_index, W_in, b_in, W_l, W_r, att,
                bias_gat, W1, b1, W2, b2, W3, b3)
